# Initial kernel scaffold; baseline (speedup 1.0000x reference)
import jax
import jax.numpy as jnp
from jax.experimental import pallas as pl


def kernel(x, adj_index, adj_values, W, b):
    raise NotImplementedError("write your pallas kernel here")


# speedup vs baseline: 3.4499x; 3.4499x over previous
import functools

import jax
import jax.numpy as jnp
from jax import lax
from jax.experimental import pallas as pl
from jax.experimental.pallas import tpu as pltpu
from jax.experimental.pallas import tpu_sc as plsc

N = 10000
E = 320000
D_IN = 128
D_OUT = 128
HALF = 64
NC = 2
NT = 16
LANES = 16
CHUNK = 1024
SUB = 128
GSUB = CHUNK // SUB
E_PAD = 327680
EDGES_PER_TILE = E_PAD // NT
CHUNKS_PER_TILE = EDGES_PER_TILE // CHUNK
ROWS_A = 624
ROWS_B = 640


def _matmul_body(x_ref, w_ref, o_ref):
    o_ref[...] = jnp.dot(x_ref[...], w_ref[0],
                         preferred_element_type=jnp.float32)


def _stacked_matmul(x, W):
    BLK = 400
    nb = N // BLK
    Wh = jnp.stack([W[:, :HALF], W[:, HALF:]])
    return pl.pallas_call(
        _matmul_body,
        grid=(NC, nb),
        in_specs=[
            pl.BlockSpec((BLK, D_IN), lambda c, i: (i, 0)),
            pl.BlockSpec((1, D_IN, HALF), lambda c, i: (c, 0, 0)),
        ],
        out_specs=pl.BlockSpec((BLK, HALF), lambda c, i: (c * nb + i, 0)),
        out_shape=jax.ShapeDtypeStruct((NC * N, HALF), jnp.float32),
    )(x, Wh)


def _spmm_sc(m2, src_p, dst_p, val_p):
    mesh = plsc.VectorSubcoreMesh(core_axis_name="c", subcore_axis_name="s")

    @functools.partial(
        pl.kernel,
        out_type=jax.ShapeDtypeStruct((NC, N, HALF), jnp.float32),
        mesh=mesh,
        compiler_params=pltpu.CompilerParams(use_tc_tiling_on_sc=False),
        scratch_types=[
            pltpu.VMEM((CHUNK,), jnp.int32),
            pltpu.VMEM((GSUB, SUB), jnp.int32),
            pltpu.VMEM((CHUNK,), jnp.int32),
            pltpu.VMEM((GSUB, SUB), jnp.int32),
            pltpu.VMEM((CHUNK,), jnp.float32),
            pltpu.VMEM((CHUNK, HALF), jnp.float32),
            pltpu.VMEM_SHARED((N, HALF), jnp.float32),
            pltpu.SemaphoreType.DMA,
        ],
    )
    def k(m_hbm, src_hbm, dst_hbm, val_hbm, out_hbm,
          srcv, src2, dstv, dst2, valv, rows, acc, sem):
        c = lax.axis_index("c")
        s = lax.axis_index("s")

        def zrow(i, carry):
            z = jnp.zeros((LANES,), jnp.float32)
            for j in range(HALF // LANES):
                rows[i, pl.ds(j * LANES, LANES)] = z
            return carry
        lax.fori_loop(0, ROWS_B, zrow, 0)

        @pl.when(s < NT - 1)
        def _():
            pltpu.sync_copy(rows.at[pl.ds(0, ROWS_A)],
                            acc.at[pl.ds(s * ROWS_A, ROWS_A)])

        @pl.when(s == NT - 1)
        def _():
            pltpu.sync_copy(rows.at[pl.ds(0, ROWS_B)],
                            acc.at[pl.ds((NT - 1) * ROWS_A, ROWS_B)])
        plsc.subcore_barrier()

        base0 = s * EDGES_PER_TILE
        half_off = jnp.full((LANES,), 1, jnp.int32) * (c * N)

        def chunk_body(kk, carry):
            base = base0 + kk * CHUNK
            pltpu.sync_copy(src_hbm.at[pl.ds(base, CHUNK)], srcv)
            pltpu.sync_copy(dst_hbm.at[pl.ds(base, CHUNK)], dstv)
            pltpu.sync_copy(val_hbm.at[pl.ds(base, CHUNK)], valv)

            def cp(i, cc):
                g = i // (SUB // LANES)
                r = i % (SUB // LANES)
                sl = pl.ds(r * LANES, LANES)
                src2[g, sl] = srcv[pl.ds(i * LANES, LANES)] + half_off
                dst2[g, sl] = dstv[pl.ds(i * LANES, LANES)]
                return cc
            lax.fori_loop(0, CHUNK // LANES, cp, 0)

            cps = [pltpu.async_copy(m_hbm.at[src2.at[g]],
                                    rows.at[pl.ds(g * SUB, SUB)], sem)
                   for g in range(GSUB)]
            for cpd in cps:
                cpd.wait()

            def scale(i, cc):
                vv = valv[pl.ds(i * LANES, LANES)]
                for l in range(LANES):
                    r = i * LANES + l
                    v = vv[l]
                    for j in range(HALF // LANES):
                        sl = pl.ds(j * LANES, LANES)
                        rows[r, sl] = rows[r, sl] * v
                return cc
            lax.fori_loop(0, CHUNK // LANES, scale, 0)

            for g in range(GSUB):
                pltpu.sync_copy(rows.at[pl.ds(g * SUB, SUB)],
                                acc.at[dst2.at[g]], add=True)
            return carry
        lax.fori_loop(0, CHUNKS_PER_TILE, chunk_body, 0)

        plsc.subcore_barrier()

        @pl.when(s < NT - 1)
        def _():
            pltpu.sync_copy(acc.at[pl.ds(s * ROWS_A, ROWS_A)],
                            out_hbm.at[c, pl.ds(s * ROWS_A, ROWS_A)])

        @pl.when(s == NT - 1)
        def _():
            pltpu.sync_copy(acc.at[pl.ds((NT - 1) * ROWS_A, ROWS_B)],
                            out_hbm.at[c, pl.ds((NT - 1) * ROWS_A, ROWS_B)])

    return k(m2, src_p, dst_p, val_p)


def kernel(x, adj_index, adj_values, W, b):
    m2 = _stacked_matmul(x, W)
    src = adj_index[1]
    dst = adj_index[0]
    pad = E_PAD - E
    pad_idx = (jnp.arange(pad, dtype=jnp.int32) * 13) % N
    src_p = jnp.concatenate([src, pad_idx])
    dst_p = jnp.concatenate([dst, pad_idx])
    val_p = jnp.concatenate([adj_values, jnp.zeros((pad,), jnp.float32)])
    out2 = _spmm_sc(m2, src_p, dst_p, val_p)
    return jnp.concatenate([out2[0], out2[1]], axis=1) + b


# speedup vs baseline: 5.1588x; 1.4953x over previous
import functools

import jax
import jax.numpy as jnp
from jax import lax
from jax.experimental import pallas as pl
from jax.experimental.pallas import tpu as pltpu
from jax.experimental.pallas import tpu_sc as plsc

N = 10000
E = 320000
D_IN = 128
D_OUT = 128
HALF = 64
NC = 2
NT = 16
LANES = 16
CHUNK = 640
SUB = 128
GSUB = CHUNK // SUB
E_PAD = 327680
EDGES_PER_TILE = E_PAD // NT
NCH = EDGES_PER_TILE // CHUNK
ROWS_A = 624
ROWS_B = 640


def _matmul_body(x_ref, w_ref, o_ref):
    o_ref[...] = jnp.dot(x_ref[...], w_ref[0],
                         preferred_element_type=jnp.float32)


def _stacked_matmul(x, W):
    BLK = 400
    nb = N // BLK
    Wh = jnp.stack([W[:, :HALF], W[:, HALF:]])
    return pl.pallas_call(
        _matmul_body,
        grid=(NC, nb),
        in_specs=[
            pl.BlockSpec((BLK, D_IN), lambda c, i: (i, 0)),
            pl.BlockSpec((1, D_IN, HALF), lambda c, i: (c, 0, 0)),
        ],
        out_specs=pl.BlockSpec((BLK, HALF), lambda c, i: (c * nb + i, 0)),
        out_shape=jax.ShapeDtypeStruct((NC * N, HALF), jnp.float32),
    )(x, Wh)


def _spmm_sc(m2, src_p, dst_p, val_p):
    mesh = plsc.VectorSubcoreMesh(core_axis_name="c", subcore_axis_name="s")

    @functools.partial(
        pl.kernel,
        out_type=jax.ShapeDtypeStruct((NC, N, HALF), jnp.float32),
        mesh=mesh,
        compiler_params=pltpu.CompilerParams(use_tc_tiling_on_sc=False),
        scratch_types=[
            pltpu.VMEM((2, CHUNK), jnp.int32),
            pltpu.VMEM((2, GSUB, SUB), jnp.int32),
            pltpu.VMEM((2, CHUNK), jnp.int32),
            pltpu.VMEM((2, GSUB, SUB), jnp.int32),
            pltpu.VMEM((2, CHUNK), jnp.float32),
            pltpu.VMEM((2, CHUNK, HALF), jnp.float32),
            pltpu.VMEM_SHARED((N, HALF), jnp.float32),
            pltpu.SemaphoreType.DMA,
            pltpu.SemaphoreType.DMA,
            pltpu.SemaphoreType.DMA,
            pltpu.SemaphoreType.DMA,
            pltpu.SemaphoreType.DMA,
            pltpu.SemaphoreType.DMA,
        ],
    )
    def k(m_hbm, src_hbm, dst_hbm, val_hbm, out_hbm,
          srcv, src2, dstv, dst2, valv, rows, acc,
          sem_g0, sem_g1, sem_i0, sem_i1, sem_s0, sem_s1):
        c = lax.axis_index("c")
        s = lax.axis_index("s")
        sem_g = (sem_g0, sem_g1)
        sem_i = (sem_i0, sem_i1)
        sem_s = (sem_s0, sem_s1)

        def zrow(i, carry):
            z = jnp.zeros((LANES,), jnp.float32)
            for j in range(HALF // LANES):
                rows[0, i, pl.ds(j * LANES, LANES)] = z
            return carry
        lax.fori_loop(0, ROWS_B, zrow, 0)

        @pl.when(s < NT - 1)
        def _():
            pltpu.sync_copy(rows.at[0, pl.ds(0, ROWS_A)],
                            acc.at[pl.ds(s * ROWS_A, ROWS_A)])

        @pl.when(s == NT - 1)
        def _():
            pltpu.sync_copy(rows.at[0, pl.ds(0, ROWS_B)],
                            acc.at[pl.ds((NT - 1) * ROWS_A, ROWS_B)])
        plsc.subcore_barrier()

        base0 = s * EDGES_PER_TILE
        half_off = jnp.full((LANES,), 1, jnp.int32) * (c * N)

        def fire_idx(kk, p):
            base = base0 + kk * CHUNK
            pltpu.async_copy(src_hbm.at[pl.ds(base, CHUNK)], srcv.at[p],
                             sem_i[p])
            pltpu.async_copy(dst_hbm.at[pl.ds(base, CHUNK)], dstv.at[p],
                             sem_i[p])
            pltpu.async_copy(val_hbm.at[pl.ds(base, CHUNK)], valv.at[p],
                             sem_i[p])

        def wait_idx(p):
            pltpu.make_async_copy(src_hbm.at[pl.ds(0, CHUNK)], srcv.at[p],
                                  sem_i[p]).wait()
            pltpu.make_async_copy(dst_hbm.at[pl.ds(0, CHUNK)], dstv.at[p],
                                  sem_i[p]).wait()
            pltpu.make_async_copy(val_hbm.at[pl.ds(0, CHUNK)], valv.at[p],
                                  sem_i[p]).wait()

        def build_idx(p):
            def cp(i, cc):
                g = i // (SUB // LANES)
                r = i % (SUB // LANES)
                sl = pl.ds(r * LANES, LANES)
                src2[p, g, sl] = srcv[p, pl.ds(i * LANES, LANES)] + half_off
                dst2[p, g, sl] = dstv[p, pl.ds(i * LANES, LANES)]
                return cc
            lax.fori_loop(0, CHUNK // LANES, cp, 0)

        def fire_gather(p):
            for g in range(GSUB):
                pltpu.async_copy(m_hbm.at[src2.at[p, g]],
                                 rows.at[p, pl.ds(g * SUB, SUB)], sem_g[p])

        def wait_gather(p):
            pltpu.make_async_copy(m_hbm.at[pl.ds(0, CHUNK)], rows.at[p],
                                  sem_g[p]).wait()

        def fire_scatter(p):
            for g in range(GSUB):
                pltpu.async_copy(rows.at[p, pl.ds(g * SUB, SUB)],
                                 acc.at[dst2.at[p, g]], sem_s[p], add=True)

        def wait_scatter(p):
            pltpu.make_async_copy(m_hbm.at[pl.ds(0, CHUNK)], rows.at[p],
                                  sem_s[p]).wait()

        def scale(p):
            def body(i, cc):
                vv = valv[p, pl.ds(i * LANES, LANES)]
                for l in range(LANES):
                    r = i * LANES + l
                    v = vv[l]
                    for j in range(HALF // LANES):
                        sl = pl.ds(j * LANES, LANES)
                        rows[p, r, sl] = rows[p, r, sl] * v
                return cc
            lax.fori_loop(0, CHUNK // LANES, body, 0)

        fire_idx(0, 0)
        wait_idx(0)
        build_idx(0)
        fire_gather(0)
        fire_idx(1, 1)

        def half_step(kk, p):
            q = 1 - p
            wait_gather(p)
            scale(p)
            fire_scatter(p)

            @pl.when(kk + 1 < NCH)
            def _():
                wait_idx(q)
                build_idx(q)

            @pl.when(kk >= 1)
            def _():
                wait_scatter(q)

            @pl.when(kk + 1 < NCH)
            def _():
                fire_gather(q)

            @pl.when(kk + 2 < NCH)
            def _():
                fire_idx(kk + 2, p)

        def pair_body(j, carry):
            half_step(2 * j, 0)
            half_step(2 * j + 1, 1)
            return carry
        lax.fori_loop(0, NCH // 2, pair_body, 0)

        wait_scatter(1)

        plsc.subcore_barrier()

        @pl.when(s < NT - 1)
        def _():
            pltpu.sync_copy(acc.at[pl.ds(s * ROWS_A, ROWS_A)],
                            out_hbm.at[c, pl.ds(s * ROWS_A, ROWS_A)])

        @pl.when(s == NT - 1)
        def _():
            pltpu.sync_copy(acc.at[pl.ds((NT - 1) * ROWS_A, ROWS_B)],
                            out_hbm.at[c, pl.ds((NT - 1) * ROWS_A, ROWS_B)])

    return k(m2, src_p, dst_p, val_p)


def kernel(x, adj_index, adj_values, W, b):
    m2 = _stacked_matmul(x, W)
    src = adj_index[1]
    dst = adj_index[0]
    pad = E_PAD - E
    pad_idx = (jnp.arange(pad, dtype=jnp.int32) * 13) % N
    src_p = jnp.concatenate([src, pad_idx])
    dst_p = jnp.concatenate([dst, pad_idx])
    val_p = jnp.concatenate([adj_values, jnp.zeros((pad,), jnp.float32)])
    out2 = _spmm_sc(m2, src_p, dst_p, val_p)
    return jnp.concatenate([out2[0], out2[1]], axis=1) + b


# speedup vs baseline: 8.1489x; 1.5796x over previous
import functools

import jax
import jax.numpy as jnp
from jax import lax
from jax.experimental import pallas as pl
from jax.experimental.pallas import tpu as pltpu
from jax.experimental.pallas import tpu_sc as plsc

N = 10000
E = 320000
D_IN = 128
D_OUT = 128
HALF = 64
NC = 2
NT = 16
LANES = 16
CHUNK = 640
SUB = 128
GSUB = CHUNK // SUB
E_PAD = 327680
EDGES_PER_TILE = E_PAD // NT
NCH = EDGES_PER_TILE // CHUNK
ROWS_A = 624
ROWS_B = 640


def _matmul_body(x_ref, w_ref, o_ref):
    o_ref[...] = jnp.dot(x_ref[...], w_ref[0],
                         preferred_element_type=jnp.float32)


def _stacked_matmul(x, W):
    BLK = 400
    nb = N // BLK
    Wh = jnp.stack([W[:, :HALF], W[:, HALF:]])
    return pl.pallas_call(
        _matmul_body,
        grid=(NC, nb),
        in_specs=[
            pl.BlockSpec((BLK, D_IN), lambda c, i: (i, 0)),
            pl.BlockSpec((1, D_IN, HALF), lambda c, i: (c, 0, 0)),
        ],
        out_specs=pl.BlockSpec((BLK, HALF), lambda c, i: (c * nb + i, 0)),
        out_shape=jax.ShapeDtypeStruct((NC * N, HALF), jnp.float32),
    )(x, Wh)


def _spmm_sc(m2, src_p, dst_p, val_p):
    mesh = plsc.VectorSubcoreMesh(core_axis_name="c", subcore_axis_name="s")

    @functools.partial(
        pl.kernel,
        out_type=jax.ShapeDtypeStruct((NC, N, HALF), jnp.float32),
        mesh=mesh,
        compiler_params=pltpu.CompilerParams(use_tc_tiling_on_sc=False),
        scratch_types=[
            pltpu.VMEM((2, CHUNK), jnp.int32),
            pltpu.VMEM((2, GSUB, SUB), jnp.int32),
            pltpu.VMEM((2, CHUNK), jnp.int32),
            pltpu.VMEM((2, GSUB, SUB), jnp.int32),
            pltpu.VMEM((2, CHUNK), jnp.float32),
            pltpu.VMEM((2, CHUNK, HALF), jnp.float32),
            pltpu.VMEM_SHARED((N, HALF), jnp.float32),
            pltpu.SemaphoreType.DMA,
            pltpu.SemaphoreType.DMA,
            pltpu.SemaphoreType.DMA,
            pltpu.SemaphoreType.DMA,
            pltpu.SemaphoreType.DMA,
            pltpu.SemaphoreType.DMA,
        ],
    )
    def k(m_hbm, src_hbm, dst_hbm, val_hbm, out_hbm,
          srcv, src2, dstv, dst2, valv, rows, acc,
          sem_g0, sem_g1, sem_i0, sem_i1, sem_s0, sem_s1):
        c = lax.axis_index("c")
        s = lax.axis_index("s")
        sem_g = (sem_g0, sem_g1)
        sem_i = (sem_i0, sem_i1)
        sem_s = (sem_s0, sem_s1)

        @plsc.parallel_loop(0, ROWS_B, unroll=4)
        def _(i):
            z = jnp.zeros((LANES,), jnp.float32)
            for j in range(HALF // LANES):
                rows[0, i, pl.ds(j * LANES, LANES)] = z

        @pl.when(s < NT - 1)
        def _():
            pltpu.sync_copy(rows.at[0, pl.ds(0, ROWS_A)],
                            acc.at[pl.ds(s * ROWS_A, ROWS_A)])

        @pl.when(s == NT - 1)
        def _():
            pltpu.sync_copy(rows.at[0, pl.ds(0, ROWS_B)],
                            acc.at[pl.ds((NT - 1) * ROWS_A, ROWS_B)])
        plsc.subcore_barrier()

        base0 = s * EDGES_PER_TILE
        half_off = jnp.full((LANES,), 1, jnp.int32) * (c * N)

        def fire_idx(kk, p):
            base = base0 + kk * CHUNK
            pltpu.async_copy(src_hbm.at[pl.ds(base, CHUNK)], srcv.at[p],
                             sem_i[p])
            pltpu.async_copy(dst_hbm.at[pl.ds(base, CHUNK)], dstv.at[p],
                             sem_i[p])
            pltpu.async_copy(val_hbm.at[pl.ds(base, CHUNK)], valv.at[p],
                             sem_i[p])

        def wait_idx(p):
            pltpu.make_async_copy(src_hbm.at[pl.ds(0, CHUNK)], srcv.at[p],
                                  sem_i[p]).wait()
            pltpu.make_async_copy(dst_hbm.at[pl.ds(0, CHUNK)], dstv.at[p],
                                  sem_i[p]).wait()
            pltpu.make_async_copy(val_hbm.at[pl.ds(0, CHUNK)], valv.at[p],
                                  sem_i[p]).wait()

        def build_idx(p):
            @plsc.parallel_loop(0, CHUNK // LANES, unroll=4)
            def _(i):
                g = i // (SUB // LANES)
                r = i % (SUB // LANES)
                sl = pl.ds(r * LANES, LANES)
                src2[p, g, sl] = srcv[p, pl.ds(i * LANES, LANES)] + half_off
                dst2[p, g, sl] = dstv[p, pl.ds(i * LANES, LANES)]

        def fire_gather(p):
            for g in range(GSUB):
                pltpu.async_copy(m_hbm.at[src2.at[p, g]],
                                 rows.at[p, pl.ds(g * SUB, SUB)], sem_g[p])

        def wait_gather(p):
            pltpu.make_async_copy(m_hbm.at[pl.ds(0, CHUNK)], rows.at[p],
                                  sem_g[p]).wait()

        def fire_scatter(p):
            for g in range(GSUB):
                pltpu.async_copy(rows.at[p, pl.ds(g * SUB, SUB)],
                                 acc.at[dst2.at[p, g]], sem_s[p], add=True)

        def wait_scatter(p):
            pltpu.make_async_copy(m_hbm.at[pl.ds(0, CHUNK)], rows.at[p],
                                  sem_s[p]).wait()

        def scale(p):
            @plsc.parallel_loop(0, CHUNK // LANES, unroll=2)
            def _(i):
                vv = valv[p, pl.ds(i * LANES, LANES)]
                for l in range(LANES):
                    r = i * LANES + l
                    v = vv[l]
                    for j in range(HALF // LANES):
                        sl = pl.ds(j * LANES, LANES)
                        rows[p, r, sl] = rows[p, r, sl] * v

        fire_idx(0, 0)
        wait_idx(0)
        build_idx(0)
        fire_gather(0)
        fire_idx(1, 1)

        def half_step(kk, p):
            q = 1 - p
            wait_gather(p)
            scale(p)
            fire_scatter(p)

            @pl.when(kk + 1 < NCH)
            def _():
                wait_idx(q)
                build_idx(q)

            @pl.when(kk >= 1)
            def _():
                wait_scatter(q)

            @pl.when(kk + 1 < NCH)
            def _():
                fire_gather(q)

            @pl.when(kk + 2 < NCH)
            def _():
                fire_idx(kk + 2, p)

        def pair_body(j, carry):
            half_step(2 * j, 0)
            half_step(2 * j + 1, 1)
            return carry
        lax.fori_loop(0, NCH // 2, pair_body, 0)

        wait_scatter(1)

        plsc.subcore_barrier()

        @pl.when(s < NT - 1)
        def _():
            pltpu.sync_copy(acc.at[pl.ds(s * ROWS_A, ROWS_A)],
                            out_hbm.at[c, pl.ds(s * ROWS_A, ROWS_A)])

        @pl.when(s == NT - 1)
        def _():
            pltpu.sync_copy(acc.at[pl.ds((NT - 1) * ROWS_A, ROWS_B)],
                            out_hbm.at[c, pl.ds((NT - 1) * ROWS_A, ROWS_B)])

    return k(m2, src_p, dst_p, val_p)


def kernel(x, adj_index, adj_values, W, b):
    m2 = _stacked_matmul(x, W)
    src = adj_index[1]
    dst = adj_index[0]
    pad = E_PAD - E
    pad_idx = (jnp.arange(pad, dtype=jnp.int32) * 13) % N
    src_p = jnp.concatenate([src, pad_idx])
    dst_p = jnp.concatenate([dst, pad_idx])
    val_p = jnp.concatenate([adj_values, jnp.zeros((pad,), jnp.float32)])
    out2 = _spmm_sc(m2, src_p, dst_p, val_p)
    return jnp.concatenate([out2[0], out2[1]], axis=1) + b


# speedup vs baseline: 8.7195x; 1.0700x over previous
import functools

import jax
import jax.numpy as jnp
from jax import lax
from jax.experimental import pallas as pl
from jax.experimental.pallas import tpu as pltpu
from jax.experimental.pallas import tpu_sc as plsc

N = 10000
E = 320000
D_IN = 128
D_OUT = 128
HALF = 64
NC = 2
NT = 16
LANES = 16
CHUNK = 640
SUB = 128
GSUB = CHUNK // SUB
E_PAD = 327680
EDGES_PER_TILE = E_PAD // NT
NCH = EDGES_PER_TILE // CHUNK
ROWS_A = 624
ROWS_B = 640


def _matmul_body(x_ref, w_ref, o_ref):
    o_ref[...] = jnp.dot(x_ref[...], w_ref[0],
                         preferred_element_type=jnp.float32)


def _stacked_matmul(x, W):
    BLK = 400
    nb = N // BLK
    Wh = jnp.stack([W[:, :HALF], W[:, HALF:]])
    return pl.pallas_call(
        _matmul_body,
        grid=(NC, nb),
        in_specs=[
            pl.BlockSpec((BLK, D_IN), lambda c, i: (i, 0)),
            pl.BlockSpec((1, D_IN, HALF), lambda c, i: (c, 0, 0)),
        ],
        out_specs=pl.BlockSpec((BLK, HALF), lambda c, i: (c * nb + i, 0)),
        out_shape=jax.ShapeDtypeStruct((NC * N, HALF), jnp.float32),
    )(x, Wh)


def _spmm_sc(m2, src_p, dst_p, val_p, b2):
    mesh = plsc.VectorSubcoreMesh(core_axis_name="c", subcore_axis_name="s")

    @functools.partial(
        pl.kernel,
        out_type=jax.ShapeDtypeStruct((N, D_OUT), jnp.float32),
        mesh=mesh,
        compiler_params=pltpu.CompilerParams(use_tc_tiling_on_sc=False),
        scratch_types=[
            pltpu.VMEM((2, CHUNK), jnp.int32),
            pltpu.VMEM((2, GSUB, SUB), jnp.int32),
            pltpu.VMEM((2, CHUNK), jnp.int32),
            pltpu.VMEM((2, GSUB, SUB), jnp.int32),
            pltpu.VMEM((2, CHUNK), jnp.float32),
            pltpu.VMEM((2, CHUNK, HALF), jnp.float32),
            pltpu.VMEM((HALF,), jnp.float32),
            pltpu.VMEM_SHARED((N, HALF), jnp.float32),
            pltpu.SemaphoreType.DMA,
            pltpu.SemaphoreType.DMA,
            pltpu.SemaphoreType.DMA,
            pltpu.SemaphoreType.DMA,
            pltpu.SemaphoreType.DMA,
            pltpu.SemaphoreType.DMA,
        ],
    )
    def k(m_hbm, src_hbm, dst_hbm, val_hbm, b_hbm, out_hbm,
          srcv, src2, dstv, dst2, valv, rows, bb, acc,
          sem_g0, sem_g1, sem_i0, sem_i1, sem_s0, sem_s1):
        c = lax.axis_index("c")
        s = lax.axis_index("s")
        sem_g = (sem_g0, sem_g1)
        sem_i = (sem_i0, sem_i1)
        sem_s = (sem_s0, sem_s1)

        pltpu.sync_copy(b_hbm.at[c], bb)
        bvec = [bb[pl.ds(j * LANES, LANES)] for j in range(HALF // LANES)]

        @plsc.parallel_loop(0, ROWS_B, unroll=4)
        def _(i):
            for j in range(HALF // LANES):
                rows[0, i, pl.ds(j * LANES, LANES)] = bvec[j]

        @pl.when(s < NT - 1)
        def _():
            pltpu.sync_copy(rows.at[0, pl.ds(0, ROWS_A)],
                            acc.at[pl.ds(s * ROWS_A, ROWS_A)])

        @pl.when(s == NT - 1)
        def _():
            pltpu.sync_copy(rows.at[0, pl.ds(0, ROWS_B)],
                            acc.at[pl.ds((NT - 1) * ROWS_A, ROWS_B)])
        plsc.subcore_barrier()

        base0 = s * EDGES_PER_TILE
        half_off = jnp.full((LANES,), 1, jnp.int32) * (c * N)

        def fire_idx(kk, p):
            base = base0 + kk * CHUNK
            pltpu.async_copy(src_hbm.at[pl.ds(base, CHUNK)], srcv.at[p],
                             sem_i[p])
            pltpu.async_copy(dst_hbm.at[pl.ds(base, CHUNK)], dstv.at[p],
                             sem_i[p])
            pltpu.async_copy(val_hbm.at[pl.ds(base, CHUNK)], valv.at[p],
                             sem_i[p])

        def wait_idx(p):
            pltpu.make_async_copy(src_hbm.at[pl.ds(0, CHUNK)], srcv.at[p],
                                  sem_i[p]).wait()
            pltpu.make_async_copy(dst_hbm.at[pl.ds(0, CHUNK)], dstv.at[p],
                                  sem_i[p]).wait()
            pltpu.make_async_copy(val_hbm.at[pl.ds(0, CHUNK)], valv.at[p],
                                  sem_i[p]).wait()

        def build_idx(p):
            @plsc.parallel_loop(0, CHUNK // LANES, unroll=4)
            def _(i):
                g = i // (SUB // LANES)
                r = i % (SUB // LANES)
                sl = pl.ds(r * LANES, LANES)
                src2[p, g, sl] = srcv[p, pl.ds(i * LANES, LANES)] + half_off
                dst2[p, g, sl] = dstv[p, pl.ds(i * LANES, LANES)]

        def fire_gather(p):
            for g in range(GSUB):
                pltpu.async_copy(m_hbm.at[src2.at[p, g]],
                                 rows.at[p, pl.ds(g * SUB, SUB)], sem_g[p])

        def wait_gather(p):
            pltpu.make_async_copy(m_hbm.at[pl.ds(0, CHUNK)], rows.at[p],
                                  sem_g[p]).wait()

        def fire_scatter(p):
            for g in range(GSUB):
                pltpu.async_copy(rows.at[p, pl.ds(g * SUB, SUB)],
                                 acc.at[dst2.at[p, g]], sem_s[p], add=True)

        def wait_scatter(p):
            pltpu.make_async_copy(m_hbm.at[pl.ds(0, CHUNK)], rows.at[p],
                                  sem_s[p]).wait()

        def scale(p):
            @plsc.parallel_loop(0, CHUNK // LANES, unroll=4)
            def _(i):
                vv = valv[p, pl.ds(i * LANES, LANES)]
                for l in range(LANES):
                    r = i * LANES + l
                    v = vv[l]
                    for j in range(HALF // LANES):
                        sl = pl.ds(j * LANES, LANES)
                        rows[p, r, sl] = rows[p, r, sl] * v

        fire_idx(0, 0)
        wait_idx(0)
        build_idx(0)
        fire_gather(0)
        fire_idx(1, 1)

        def half_step(kk, p):
            q = 1 - p
            wait_gather(p)
            scale(p)
            fire_scatter(p)

            @pl.when(kk + 1 < NCH)
            def _():
                wait_idx(q)
                build_idx(q)

            @pl.when(kk >= 1)
            def _():
                wait_scatter(q)

            @pl.when(kk + 1 < NCH)
            def _():
                fire_gather(q)

            @pl.when(kk + 2 < NCH)
            def _():
                fire_idx(kk + 2, p)

        def pair_body(j, carry):
            half_step(2 * j, 0)
            half_step(2 * j + 1, 1)
            return carry
        lax.fori_loop(0, NCH // 2, pair_body, 0)

        wait_scatter(1)

        plsc.subcore_barrier()

        @pl.when(s < NT - 1)
        def _():
            pltpu.sync_copy(
                acc.at[pl.ds(s * ROWS_A, ROWS_A)],
                out_hbm.at[pl.ds(s * ROWS_A, ROWS_A), pl.ds(c * HALF, HALF)])

        @pl.when(s == NT - 1)
        def _():
            pltpu.sync_copy(
                acc.at[pl.ds((NT - 1) * ROWS_A, ROWS_B)],
                out_hbm.at[pl.ds((NT - 1) * ROWS_A, ROWS_B),
                           pl.ds(c * HALF, HALF)])

    return k(m2, src_p, dst_p, val_p, b2)


def kernel(x, adj_index, adj_values, W, b):
    m2 = _stacked_matmul(x, W)
    src = adj_index[1]
    dst = adj_index[0]
    pad = E_PAD - E
    pad_idx = (jnp.arange(pad, dtype=jnp.int32) * 13) % N
    src_p = jnp.concatenate([src, pad_idx])
    dst_p = jnp.concatenate([dst, pad_idx])
    val_p = jnp.concatenate([adj_values, jnp.zeros((pad,), jnp.float32)])
    b2 = b.reshape(NC, HALF)
    return _spmm_sc(m2, src_p, dst_p, val_p, b2)


# speedup vs baseline: 10.8876x; 1.2487x over previous
import functools

import jax
import jax.numpy as jnp
from jax import lax
from jax.experimental import pallas as pl
from jax.experimental.pallas import tpu as pltpu
from jax.experimental.pallas import tpu_sc as plsc

N = 10000
E = 320000
D_IN = 128
D_OUT = 128
HALF = 64
NC = 2
NT = 16
LANES = 16
NBUF = 3
CHUNK = 384
SUB = 128
GSUB = CHUNK // SUB
EDGES_PER_TILE = E // NT
NCH = 53
LAST_BASE = EDGES_PER_TILE - CHUNK
OVERLAP = NCH * CHUNK - EDGES_PER_TILE
ROWS_A = 624
ROWS_B = 640


def _matmul_body(x_ref, w_ref, o_ref):
    o_ref[...] = jnp.dot(x_ref[...], w_ref[0],
                         preferred_element_type=jnp.float32)


def _stacked_matmul(x, W):
    BLK = 400
    nb = N // BLK
    Wh = jnp.stack([W[:, :HALF], W[:, HALF:]])
    return pl.pallas_call(
        _matmul_body,
        grid=(NC, nb),
        in_specs=[
            pl.BlockSpec((BLK, D_IN), lambda c, i: (i, 0)),
            pl.BlockSpec((1, D_IN, HALF), lambda c, i: (c, 0, 0)),
        ],
        out_specs=pl.BlockSpec((BLK, HALF), lambda c, i: (c * nb + i, 0)),
        out_shape=jax.ShapeDtypeStruct((NC * N, HALF), jnp.float32),
    )(x, Wh)


def _spmm_sc(m2, adj_index, adj_values, b2):
    mesh = plsc.VectorSubcoreMesh(core_axis_name="c", subcore_axis_name="s")

    @functools.partial(
        pl.kernel,
        out_type=jax.ShapeDtypeStruct((N, D_OUT), jnp.float32),
        mesh=mesh,
        compiler_params=pltpu.CompilerParams(use_tc_tiling_on_sc=False),
        scratch_types=[
            pltpu.VMEM((NBUF, CHUNK), jnp.int32),
            pltpu.VMEM((NBUF, GSUB, SUB), jnp.int32),
            pltpu.VMEM((NBUF, CHUNK), jnp.int32),
            pltpu.VMEM((NBUF, GSUB, SUB), jnp.int32),
            pltpu.VMEM((NBUF, CHUNK), jnp.float32),
            pltpu.VMEM((NBUF, CHUNK, HALF), jnp.float32),
            pltpu.VMEM((HALF,), jnp.float32),
            pltpu.VMEM_SHARED((N, HALF), jnp.float32),
            [pltpu.SemaphoreType.DMA] * NBUF,
            [pltpu.SemaphoreType.DMA] * NBUF,
            [pltpu.SemaphoreType.DMA] * NBUF,
        ],
    )
    def k(m_hbm, adj_hbm, val_hbm, b_hbm, out_hbm,
          srcv, src2, dstv, dst2, valv, rows, bb, acc,
          sem_g, sem_i, sem_s):
        c = lax.axis_index("c")
        s = lax.axis_index("s")

        pltpu.sync_copy(b_hbm.at[c], bb)
        bvec = [bb[pl.ds(j * LANES, LANES)] for j in range(HALF // LANES)]

        @plsc.parallel_loop(0, CHUNK, unroll=4)
        def _(i):
            for j in range(HALF // LANES):
                rows[0, i, pl.ds(j * LANES, LANES)] = bvec[j]

        @plsc.parallel_loop(0, ROWS_B - CHUNK, unroll=4)
        def _(i):
            for j in range(HALF // LANES):
                rows[1, i, pl.ds(j * LANES, LANES)] = bvec[j]

        @pl.when(s < NT - 1)
        def _():
            pltpu.sync_copy(rows.at[0], acc.at[pl.ds(s * ROWS_A, CHUNK)])
            pltpu.sync_copy(rows.at[1, pl.ds(0, ROWS_A - CHUNK)],
                            acc.at[pl.ds(s * ROWS_A + CHUNK, ROWS_A - CHUNK)])

        @pl.when(s == NT - 1)
        def _():
            base = (NT - 1) * ROWS_A
            pltpu.sync_copy(rows.at[0], acc.at[pl.ds(base, CHUNK)])
            pltpu.sync_copy(rows.at[1, pl.ds(0, ROWS_B - CHUNK)],
                            acc.at[pl.ds(base + CHUNK, ROWS_B - CHUNK)])
        plsc.subcore_barrier()

        base0 = s * EDGES_PER_TILE
        half_off = jnp.full((LANES,), 1, jnp.int32) * (c * N)

        def fire_idx(kk, p):
            base = base0 + jnp.minimum(kk * CHUNK, LAST_BASE)
            pltpu.async_copy(adj_hbm.at[1, pl.ds(base, CHUNK)], srcv.at[p],
                             sem_i[p])
            pltpu.async_copy(adj_hbm.at[0, pl.ds(base, CHUNK)], dstv.at[p],
                             sem_i[p])
            pltpu.async_copy(val_hbm.at[pl.ds(base, CHUNK)], valv.at[p],
                             sem_i[p])

        def wait_idx(p):
            pltpu.make_async_copy(adj_hbm.at[1, pl.ds(0, CHUNK)], srcv.at[p],
                                  sem_i[p]).wait()
            pltpu.make_async_copy(adj_hbm.at[0, pl.ds(0, CHUNK)], dstv.at[p],
                                  sem_i[p]).wait()
            pltpu.make_async_copy(val_hbm.at[pl.ds(0, CHUNK)], valv.at[p],
                                  sem_i[p]).wait()

        def zero_overlap(p):
            @plsc.parallel_loop(0, OVERLAP // LANES, unroll=4)
            def _(i):
                valv[p, pl.ds(i * LANES, LANES)] = jnp.zeros((LANES,),
                                                             jnp.float32)

        def build_idx(p):
            @plsc.parallel_loop(0, CHUNK // LANES, unroll=4)
            def _(i):
                g = i // (SUB // LANES)
                r = i % (SUB // LANES)
                sl = pl.ds(r * LANES, LANES)
                src2[p, g, sl] = srcv[p, pl.ds(i * LANES, LANES)] + half_off
                dst2[p, g, sl] = dstv[p, pl.ds(i * LANES, LANES)]

        def fire_gather(p):
            for g in range(GSUB):
                pltpu.async_copy(m_hbm.at[src2.at[p, g]],
                                 rows.at[p, pl.ds(g * SUB, SUB)], sem_g[p])

        def wait_gather(p):
            pltpu.make_async_copy(m_hbm.at[pl.ds(0, CHUNK)], rows.at[p],
                                  sem_g[p]).wait()

        def fire_scatter(p):
            for g in range(GSUB):
                pltpu.async_copy(rows.at[p, pl.ds(g * SUB, SUB)],
                                 acc.at[dst2.at[p, g]], sem_s[p], add=True)

        def wait_scatter(p):
            pltpu.make_async_copy(m_hbm.at[pl.ds(0, CHUNK)], rows.at[p],
                                  sem_s[p]).wait()

        def scale(p):
            @plsc.parallel_loop(0, CHUNK // LANES, unroll=4)
            def _(i):
                vv = valv[p, pl.ds(i * LANES, LANES)]
                for l in range(LANES):
                    r = i * LANES + l
                    v = vv[l]
                    for j in range(HALF // LANES):
                        sl = pl.ds(j * LANES, LANES)
                        rows[p, r, sl] = rows[p, r, sl] * v

        fire_idx(0, 0)
        wait_idx(0)
        build_idx(0)
        fire_gather(0)
        fire_idx(1, 1)

        def half_step(kk, p):
            p1 = (p + 1) % NBUF
            p2 = (p + 2) % NBUF

            @pl.when(kk + 1 < NCH)
            def _():
                wait_idx(p1)

                @pl.when(kk + 1 == NCH - 1)
                def _():
                    zero_overlap(p1)

            @pl.when(kk >= 2)
            def _():
                wait_scatter(p1)

            @pl.when(kk + 1 < NCH)
            def _():
                build_idx(p1)
                fire_gather(p1)

            wait_gather(p)
            scale(p)
            fire_scatter(p)

            @pl.when(kk + 2 < NCH)
            def _():
                fire_idx(kk + 2, p2)

        def triple_body(j, carry):
            half_step(3 * j, 0)
            half_step(3 * j + 1, 1)
            half_step(3 * j + 2, 2)
            return carry
        lax.fori_loop(0, NCH // NBUF, triple_body, 0)
        for kk in range(NBUF * (NCH // NBUF), NCH):
            half_step(kk, kk % NBUF)

        wait_scatter((NCH - 2) % NBUF)
        wait_scatter((NCH - 1) % NBUF)

        plsc.subcore_barrier()

        @pl.when(s < NT - 1)
        def _():
            pltpu.sync_copy(
                acc.at[pl.ds(s * ROWS_A, ROWS_A)],
                out_hbm.at[pl.ds(s * ROWS_A, ROWS_A), pl.ds(c * HALF, HALF)])

        @pl.when(s == NT - 1)
        def _():
            pltpu.sync_copy(
                acc.at[pl.ds((NT - 1) * ROWS_A, ROWS_B)],
                out_hbm.at[pl.ds((NT - 1) * ROWS_A, ROWS_B),
                           pl.ds(c * HALF, HALF)])

    return k(m2, adj_index, adj_values, b2)


def kernel(x, adj_index, adj_values, W, b):
    m2 = _stacked_matmul(x, W)
    b2 = b.reshape(NC, HALF)
    return _spmm_sc(m2, adj_index, adj_values, b2)


# speedup vs baseline: 11.8310x; 1.0866x over previous
import functools

import jax
import jax.numpy as jnp
from jax import lax
from jax.experimental import pallas as pl
from jax.experimental.pallas import tpu as pltpu
from jax.experimental.pallas import tpu_sc as plsc

N = 10000
E = 320000
D_IN = 128
D_OUT = 128
HALF = 64
NC = 2
NT = 16
LANES = 16
NBUF = 3
CHUNK = 384
SUB = 128
GSUB = CHUNK // SUB
EDGES_PER_TILE = E // NT
NCH = 53
LAST_BASE = EDGES_PER_TILE - CHUNK
OVERLAP = NCH * CHUNK - EDGES_PER_TILE
ROWS_A = 624
ROWS_B = 640


def _matmul_body(x_ref, w_ref, o_ref):
    d = jnp.dot(x_ref[...], w_ref[...], preferred_element_type=jnp.float32)
    o_ref[0] = d[:, :HALF]
    o_ref[1] = d[:, HALF:]


def _stacked_matmul(x, W):
    BLK = 400
    return pl.pallas_call(
        _matmul_body,
        grid=(N // BLK,),
        in_specs=[
            pl.BlockSpec((BLK, D_IN), lambda i: (i, 0)),
            pl.BlockSpec((D_IN, D_OUT), lambda i: (0, 0)),
        ],
        out_specs=pl.BlockSpec((NC, BLK, HALF), lambda i: (0, i, 0)),
        out_shape=jax.ShapeDtypeStruct((NC, N, HALF), jnp.float32),
    )(x, W)


def _spmm_sc(m2, adj_index, adj_values, b2):
    mesh = plsc.VectorSubcoreMesh(core_axis_name="c", subcore_axis_name="s")

    @functools.partial(
        pl.kernel,
        out_type=jax.ShapeDtypeStruct((N, D_OUT), jnp.float32),
        mesh=mesh,
        compiler_params=pltpu.CompilerParams(use_tc_tiling_on_sc=False),
        scratch_types=[
            pltpu.VMEM((NBUF, CHUNK), jnp.int32),
            pltpu.VMEM((NBUF, CHUNK), jnp.int32),
            pltpu.VMEM((NBUF, GSUB, SUB), jnp.int32),
            pltpu.VMEM((NBUF, CHUNK), jnp.float32),
            pltpu.VMEM((NBUF, CHUNK, HALF), jnp.float32),
            pltpu.VMEM((HALF,), jnp.float32),
            pltpu.VMEM_SHARED((N, HALF), jnp.float32),
            [pltpu.SemaphoreType.DMA] * NBUF,
            [pltpu.SemaphoreType.DMA] * NBUF,
            [pltpu.SemaphoreType.DMA] * NBUF,
        ],
    )
    def k(m_hbm, adj_hbm, val_hbm, b_hbm, out_hbm,
          srcv, dstv, dst2, valv, rows, bb, acc,
          sem_g, sem_i, sem_s):
        c = lax.axis_index("c")
        s = lax.axis_index("s")

        pltpu.sync_copy(b_hbm.at[c], bb)
        bvec = [bb[pl.ds(j * LANES, LANES)] for j in range(HALF // LANES)]

        @plsc.parallel_loop(0, CHUNK, unroll=4)
        def _(i):
            for j in range(HALF // LANES):
                rows[0, i, pl.ds(j * LANES, LANES)] = bvec[j]

        @plsc.parallel_loop(0, ROWS_B - CHUNK, unroll=4)
        def _(i):
            for j in range(HALF // LANES):
                rows[1, i, pl.ds(j * LANES, LANES)] = bvec[j]

        @pl.when(s < NT - 1)
        def _():
            pltpu.sync_copy(rows.at[0], acc.at[pl.ds(s * ROWS_A, CHUNK)])
            pltpu.sync_copy(rows.at[1, pl.ds(0, ROWS_A - CHUNK)],
                            acc.at[pl.ds(s * ROWS_A + CHUNK, ROWS_A - CHUNK)])

        @pl.when(s == NT - 1)
        def _():
            base = (NT - 1) * ROWS_A
            pltpu.sync_copy(rows.at[0], acc.at[pl.ds(base, CHUNK)])
            pltpu.sync_copy(rows.at[1, pl.ds(0, ROWS_B - CHUNK)],
                            acc.at[pl.ds(base + CHUNK, ROWS_B - CHUNK)])
        plsc.subcore_barrier()

        base0 = s * EDGES_PER_TILE
        half_off = jnp.full((LANES,), 1, jnp.int32) * (c * N)

        def fire_idx(kk, p):
            base = base0 + jnp.minimum(kk * CHUNK, LAST_BASE)
            pltpu.async_copy(adj_hbm.at[1, pl.ds(base, CHUNK)], srcv.at[p],
                             sem_i[p])
            pltpu.async_copy(adj_hbm.at[0, pl.ds(base, CHUNK)], dstv.at[p],
                             sem_i[p])
            pltpu.async_copy(val_hbm.at[pl.ds(base, CHUNK)], valv.at[p],
                             sem_i[p])

        def wait_idx(p):
            pltpu.make_async_copy(adj_hbm.at[1, pl.ds(0, CHUNK)], srcv.at[p],
                                  sem_i[p]).wait()
            pltpu.make_async_copy(adj_hbm.at[0, pl.ds(0, CHUNK)], dstv.at[p],
                                  sem_i[p]).wait()
            pltpu.make_async_copy(val_hbm.at[pl.ds(0, CHUNK)], valv.at[p],
                                  sem_i[p]).wait()

        def zero_overlap(p):
            @plsc.parallel_loop(0, OVERLAP // LANES, unroll=4)
            def _(i):
                valv[p, pl.ds(i * LANES, LANES)] = jnp.zeros((LANES,),
                                                             jnp.float32)

        def build_idx(p):
            @plsc.parallel_loop(0, CHUNK // LANES, unroll=4)
            def _(i):
                g = i // (SUB // LANES)
                r = i % (SUB // LANES)
                dst2[p, g, pl.ds(r * LANES, LANES)] = \
                    dstv[p, pl.ds(i * LANES, LANES)]

            @pl.when(c == 1)
            def _():
                @plsc.parallel_loop(0, CHUNK // LANES, unroll=4)
                def _(i):
                    sl = pl.ds(i * LANES, LANES)
                    srcv[p, sl] = srcv[p, sl] + half_off

        def fire_gather(p):
            for g in range(GSUB):
                pltpu.async_copy(m_hbm.at[srcv.at[p, pl.ds(g * SUB, SUB)]],
                                 rows.at[p, pl.ds(g * SUB, SUB)], sem_g[p])

        def wait_gather(p):
            pltpu.make_async_copy(m_hbm.at[pl.ds(0, CHUNK)], rows.at[p],
                                  sem_g[p]).wait()

        def fire_scatter(p):
            for g in range(GSUB):
                pltpu.async_copy(rows.at[p, pl.ds(g * SUB, SUB)],
                                 acc.at[dst2.at[p, g]], sem_s[p], add=True)

        def wait_scatter(p):
            pltpu.make_async_copy(m_hbm.at[pl.ds(0, CHUNK)], rows.at[p],
                                  sem_s[p]).wait()

        def scale(p):
            @plsc.parallel_loop(0, CHUNK // LANES, unroll=4)
            def _(i):
                vv = valv[p, pl.ds(i * LANES, LANES)]
                for l in range(LANES):
                    r = i * LANES + l
                    v = vv[l]
                    for j in range(HALF // LANES):
                        sl = pl.ds(j * LANES, LANES)
                        rows[p, r, sl] = rows[p, r, sl] * v

        fire_idx(0, 0)
        wait_idx(0)
        build_idx(0)
        fire_gather(0)
        fire_idx(1, 1)

        def half_step(kk, p):
            p1 = (p + 1) % NBUF
            p2 = (p + 2) % NBUF

            @pl.when(kk + 1 < NCH)
            def _():
                wait_idx(p1)

                @pl.when(kk + 1 == NCH - 1)
                def _():
                    zero_overlap(p1)

            @pl.when(kk >= 2)
            def _():
                wait_scatter(p1)

            @pl.when(kk + 1 < NCH)
            def _():
                build_idx(p1)
                fire_gather(p1)

            wait_gather(p)
            scale(p)
            fire_scatter(p)

            @pl.when(kk + 2 < NCH)
            def _():
                fire_idx(kk + 2, p2)

        def triple_body(j, carry):
            half_step(3 * j, 0)
            half_step(3 * j + 1, 1)
            half_step(3 * j + 2, 2)
            return carry
        lax.fori_loop(0, NCH // NBUF, triple_body, 0)
        for kk in range(NBUF * (NCH // NBUF), NCH):
            half_step(kk, kk % NBUF)

        wait_scatter((NCH - 2) % NBUF)
        wait_scatter((NCH - 1) % NBUF)

        plsc.subcore_barrier()

        @pl.when(s < NT - 1)
        def _():
            pltpu.sync_copy(
                acc.at[pl.ds(s * ROWS_A, ROWS_A)],
                out_hbm.at[pl.ds(s * ROWS_A, ROWS_A), pl.ds(c * HALF, HALF)])

        @pl.when(s == NT - 1)
        def _():
            pltpu.sync_copy(
                acc.at[pl.ds((NT - 1) * ROWS_A, ROWS_B)],
                out_hbm.at[pl.ds((NT - 1) * ROWS_A, ROWS_B),
                           pl.ds(c * HALF, HALF)])

    return k(m2, adj_index, adj_values, b2)


def kernel(x, adj_index, adj_values, W, b):
    m2 = _stacked_matmul(x, W).reshape(NC * N, HALF)
    b2 = b.reshape(NC, HALF)
    return _spmm_sc(m2, adj_index, adj_values, b2)


# speedup vs baseline: 13.3203x; 1.1259x over previous
import functools

import jax
import jax.numpy as jnp
from jax import lax
from jax.experimental import pallas as pl
from jax.experimental.pallas import tpu as pltpu
from jax.experimental.pallas import tpu_sc as plsc

N = 10000
E = 320000
D_IN = 128
D_OUT = 128
HALF = 64
NC = 2
NT = 16
LANES = 16
NBUF = 3
CHUNK = 384
SUB = 128
GSUB = CHUNK // SUB
EDGES_PER_TILE = E // NT
NCH = 53
LAST_BASE = EDGES_PER_TILE - CHUNK
OVERLAP = NCH * CHUNK - EDGES_PER_TILE
ROWS_A = 624
ROWS_B = 640


def _matmul_body(x_ref, w_ref, o_ref):
    o_ref[...] = jnp.dot(x_ref[...], w_ref[...],
                         preferred_element_type=jnp.float32)


def _matmul(x, W):
    BLK = 2000
    return pl.pallas_call(
        _matmul_body,
        grid=(N // BLK,),
        in_specs=[
            pl.BlockSpec((BLK, D_IN), lambda i: (i, 0)),
            pl.BlockSpec((D_IN, D_OUT), lambda i: (0, 0)),
        ],
        out_specs=pl.BlockSpec((BLK, D_OUT), lambda i: (i, 0)),
        out_shape=jax.ShapeDtypeStruct((N, D_OUT), jnp.float32),
    )(x, W)


def _spmm_sc(m2, adj_index, adj_values, b2):
    mesh = plsc.VectorSubcoreMesh(core_axis_name="c", subcore_axis_name="s")

    @functools.partial(
        pl.kernel,
        out_type=jax.ShapeDtypeStruct((N, D_OUT), jnp.float32),
        mesh=mesh,
        compiler_params=pltpu.CompilerParams(use_tc_tiling_on_sc=False),
        scratch_types=[
            pltpu.VMEM((NBUF, CHUNK), jnp.int32),
            pltpu.VMEM((NBUF, CHUNK), jnp.int32),
            pltpu.VMEM((NBUF, GSUB, SUB), jnp.int32),
            pltpu.VMEM((NBUF, CHUNK), jnp.float32),
            pltpu.VMEM((NBUF, CHUNK, HALF), jnp.float32),
            pltpu.VMEM((HALF,), jnp.float32),
            pltpu.VMEM_SHARED((N, HALF), jnp.float32),
            [pltpu.SemaphoreType.DMA] * NBUF,
            [pltpu.SemaphoreType.DMA] * NBUF,
            [pltpu.SemaphoreType.DMA] * NBUF,
        ],
    )
    def k(m_hbm, adj_hbm, val_hbm, b_hbm, out_hbm,
          srcv, dstv, dst2, valv, rows, bb, acc,
          sem_g, sem_i, sem_s):
        c = lax.axis_index("c")
        s = lax.axis_index("s")

        pltpu.sync_copy(b_hbm.at[c], bb)
        bvec = [bb[pl.ds(j * LANES, LANES)] for j in range(HALF // LANES)]

        @plsc.parallel_loop(0, CHUNK, unroll=4)
        def _(i):
            for j in range(HALF // LANES):
                rows[0, i, pl.ds(j * LANES, LANES)] = bvec[j]

        @plsc.parallel_loop(0, ROWS_B - CHUNK, unroll=4)
        def _(i):
            for j in range(HALF // LANES):
                rows[1, i, pl.ds(j * LANES, LANES)] = bvec[j]

        @pl.when(s < NT - 1)
        def _():
            pltpu.sync_copy(rows.at[0], acc.at[pl.ds(s * ROWS_A, CHUNK)])
            pltpu.sync_copy(rows.at[1, pl.ds(0, ROWS_A - CHUNK)],
                            acc.at[pl.ds(s * ROWS_A + CHUNK, ROWS_A - CHUNK)])

        @pl.when(s == NT - 1)
        def _():
            base = (NT - 1) * ROWS_A
            pltpu.sync_copy(rows.at[0], acc.at[pl.ds(base, CHUNK)])
            pltpu.sync_copy(rows.at[1, pl.ds(0, ROWS_B - CHUNK)],
                            acc.at[pl.ds(base + CHUNK, ROWS_B - CHUNK)])
        plsc.subcore_barrier()

        base0 = s * EDGES_PER_TILE
        cvec = jnp.full((LANES,), 1, jnp.int32) * c

        def fire_idx(kk, p):
            base = base0 + jnp.minimum(kk * CHUNK, LAST_BASE)
            pltpu.async_copy(adj_hbm.at[1, pl.ds(base, CHUNK)], srcv.at[p],
                             sem_i[p])
            pltpu.async_copy(adj_hbm.at[0, pl.ds(base, CHUNK)], dstv.at[p],
                             sem_i[p])
            pltpu.async_copy(val_hbm.at[pl.ds(base, CHUNK)], valv.at[p],
                             sem_i[p])

        def wait_idx(p):
            pltpu.make_async_copy(adj_hbm.at[1, pl.ds(0, CHUNK)], srcv.at[p],
                                  sem_i[p]).wait()
            pltpu.make_async_copy(adj_hbm.at[0, pl.ds(0, CHUNK)], dstv.at[p],
                                  sem_i[p]).wait()
            pltpu.make_async_copy(val_hbm.at[pl.ds(0, CHUNK)], valv.at[p],
                                  sem_i[p]).wait()

        def zero_overlap(p):
            @plsc.parallel_loop(0, OVERLAP // LANES, unroll=4)
            def _(i):
                valv[p, pl.ds(i * LANES, LANES)] = jnp.zeros((LANES,),
                                                             jnp.float32)

        def build_idx(p):
            @plsc.parallel_loop(0, CHUNK // LANES, unroll=4)
            def _(i):
                g = i // (SUB // LANES)
                r = i % (SUB // LANES)
                dst2[p, g, pl.ds(r * LANES, LANES)] = \
                    dstv[p, pl.ds(i * LANES, LANES)]

            @plsc.parallel_loop(0, CHUNK // LANES, unroll=4)
            def _(i):
                sl = pl.ds(i * LANES, LANES)
                srcv[p, sl] = (srcv[p, sl] << 1) + cvec

        def fire_gather(p):
            for g in range(GSUB):
                pltpu.async_copy(m_hbm.at[srcv.at[p, pl.ds(g * SUB, SUB)]],
                                 rows.at[p, pl.ds(g * SUB, SUB)], sem_g[p])

        def wait_gather(p):
            pltpu.make_async_copy(m_hbm.at[pl.ds(0, CHUNK)], rows.at[p],
                                  sem_g[p]).wait()

        def fire_scatter(p):
            for g in range(GSUB):
                pltpu.async_copy(rows.at[p, pl.ds(g * SUB, SUB)],
                                 acc.at[dst2.at[p, g]], sem_s[p], add=True)

        def wait_scatter(p):
            pltpu.make_async_copy(m_hbm.at[pl.ds(0, CHUNK)], rows.at[p],
                                  sem_s[p]).wait()

        def scale(p):
            @plsc.parallel_loop(0, CHUNK // LANES, unroll=4)
            def _(i):
                vv = valv[p, pl.ds(i * LANES, LANES)]
                for l in range(LANES):
                    r = i * LANES + l
                    v = vv[l]
                    for j in range(HALF // LANES):
                        sl = pl.ds(j * LANES, LANES)
                        rows[p, r, sl] = rows[p, r, sl] * v

        fire_idx(0, 0)
        wait_idx(0)
        build_idx(0)
        fire_gather(0)
        fire_idx(1, 1)

        def half_step(kk, p):
            p1 = (p + 1) % NBUF
            p2 = (p + 2) % NBUF

            @pl.when(kk + 1 < NCH)
            def _():
                wait_idx(p1)

                @pl.when(kk + 1 == NCH - 1)
                def _():
                    zero_overlap(p1)

            @pl.when(kk >= 2)
            def _():
                wait_scatter(p1)

            @pl.when(kk + 1 < NCH)
            def _():
                build_idx(p1)
                fire_gather(p1)

            wait_gather(p)
            scale(p)
            fire_scatter(p)

            @pl.when(kk + 2 < NCH)
            def _():
                fire_idx(kk + 2, p2)

        def triple_body(j, carry):
            half_step(3 * j, 0)
            half_step(3 * j + 1, 1)
            half_step(3 * j + 2, 2)
            return carry
        lax.fori_loop(0, NCH // NBUF, triple_body, 0)
        for kk in range(NBUF * (NCH // NBUF), NCH):
            half_step(kk, kk % NBUF)

        wait_scatter((NCH - 2) % NBUF)
        wait_scatter((NCH - 1) % NBUF)

        plsc.subcore_barrier()

        @pl.when(s < NT - 1)
        def _():
            pltpu.sync_copy(
                acc.at[pl.ds(s * ROWS_A, ROWS_A)],
                out_hbm.at[pl.ds(s * ROWS_A, ROWS_A), pl.ds(c * HALF, HALF)])

        @pl.when(s == NT - 1)
        def _():
            pltpu.sync_copy(
                acc.at[pl.ds((NT - 1) * ROWS_A, ROWS_B)],
                out_hbm.at[pl.ds((NT - 1) * ROWS_A, ROWS_B),
                           pl.ds(c * HALF, HALF)])

    return k(m2, adj_index, adj_values, b2)


def kernel(x, adj_index, adj_values, W, b):
    m2 = _matmul(x, W).reshape(NC * N, HALF)
    b2 = b.reshape(NC, HALF)
    return _spmm_sc(m2, adj_index, adj_values, b2)
